# Initial kernel scaffold; baseline (speedup 1.0000x reference)
import jax
import jax.numpy as jnp
from jax.experimental import pallas as pl


def kernel(triplets, ent_embed, rel_embed, a_w, a_b, a2_w, a2_b):
    raise NotImplementedError("write your pallas kernel here")


# speedup vs baseline: 2.4664x; 2.4664x over previous
import functools

import jax
import jax.numpy as jnp
from jax import lax
from jax.experimental import pallas as pl
from jax.experimental.pallas import tpu as pltpu
from jax.experimental.pallas import tpu_sc as plsc

L = 16
NC = 2
NS = 16
K = 128
AW = 144


def _prep_body(ent_ref, rel_ref, aw_ref, ab_ref, a2w_ref, a2b_ref, tri_ref,
               ph_ref, pt_ref, pr_ref, sh_ref, st_ref, sr_ref,
               hi_ref, ti_ref, ri_ref):
    d = aw_ref.shape[0]
    np_ = ent_ref.shape[0]
    aw = aw_ref[...]
    ent = ent_ref[...]
    rel = rel_ref[...]
    ph = jnp.dot(ent, aw[:, 0:d].T, preferred_element_type=jnp.float32)
    pt = jnp.dot(ent, aw[:, d:2 * d].T, preferred_element_type=jnp.float32)
    pr = (jnp.dot(rel, aw[:, 2 * d:3 * d].T,
                  preferred_element_type=jnp.float32) + ab_ref[...])
    a2 = a2w_ref[...]
    sh = jnp.dot(a2, ph.T, preferred_element_type=jnp.float32)
    st = jnp.dot(a2, pt.T, preferred_element_type=jnp.float32)
    sr = (jnp.dot(a2, pr.T, preferred_element_type=jnp.float32)
          + a2b_ref[...])
    ph_ref[...] = ph
    pt_ref[...] = pt
    pr_ref[...] = pr
    sh_ref[...] = sh.reshape(np_)
    st_ref[...] = st.reshape(np_)
    sr_ref[...] = sr.reshape(np_)
    tri = tri_ref[...]
    hi_ref[...] = tri[0]
    ti_ref[...] = tri[1]
    ri_ref[...] = tri[2]


def _fin_body(n, r, d, ha_ref, hw_ref, ra_ref, rw_ref, he_ref, hr_ref):
    hsum = ha_ref[...][:n, :d]
    ebs = hw_ref[...][:n, 0:1]
    ebs = jnp.where(ebs == 0.0, 1e-12, ebs)
    he = hsum / ebs
    he_ref[...] = jnp.where(he > 0.0, he, jnp.exp(he) - 1.0)
    rsum = ra_ref[...][:r, :d]
    cnt = rw_ref[...][:r, 1:2]
    hr = rsum / jnp.maximum(cnt, 1.0)
    hr_ref[...] = jnp.where(hr > 0.0, hr, jnp.exp(hr) - 1.0)


def _sc_body(np_, et, d, hidx, tidx, ridx, ph, pt, pr, sh_h, st_h, sr_h,
             ha, hw, ra, rw,
             hbuf, tbuf, rbuf, rowbuf, wtail, wbuf, sgh, sgt, sgr,
             acc, acc2, sem):
    cid = lax.axis_index("c")
    sid = lax.axis_index("s")
    zero16 = jnp.zeros((L,), jnp.float32)

    def zrow(i, c):
        for j in range(d // L):
            rowbuf[i, pl.ds(j * L, L)] = zero16
        wtail[i, pl.ds(0, L)] = zero16
        return c
    lax.fori_loop(0, K, zrow, 0)

    rpt = np_ // NS
    row0 = sid * rpt
    nfull = rpt // K
    for b in range(nfull):
        pltpu.sync_copy(rowbuf, acc.at[pl.ds(row0 + b * K, K)])
        pltpu.sync_copy(wtail, acc2.at[pl.ds(row0 + b * K, K)])
    rem = rpt - nfull * K
    if rem:
        pltpu.sync_copy(rowbuf.at[pl.ds(0, rem)],
                        acc.at[pl.ds(row0 + nfull * K, rem)])
        pltpu.sync_copy(wtail.at[pl.ds(0, rem)],
                        acc2.at[pl.ds(row0 + nfull * K, rem)])
    plsc.subcore_barrier()

    nchunks = et // K
    lane = lax.iota(jnp.int32, L)

    def chunk(ci, c):
        base = sid * et + ci * K
        pltpu.sync_copy(hidx.at[pl.ds(base, K)], hbuf)
        pltpu.sync_copy(tidx.at[pl.ds(base, K)], tbuf)
        pltpu.sync_copy(ridx.at[pl.ds(base, K)], rbuf)
        pltpu.async_copy(ph.at[hbuf], rowbuf, sem).wait()
        pltpu.async_copy(pt.at[tbuf], rowbuf, sem, add=True).wait()
        pltpu.async_copy(pr.at[rbuf], rowbuf, sem, add=True).wait()
        pltpu.async_copy(sh_h.at[hbuf], sgh, sem).wait()
        pltpu.async_copy(st_h.at[tbuf], sgt, sem).wait()
        pltpu.async_copy(sr_h.at[rbuf], sgr, sem).wait()

        def wgroup(g, c2):
            x = (sgh[pl.ds(g * L, L)] + sgt[pl.ds(g * L, L)]
                 + sgr[pl.ds(g * L, L)])
            wbuf[pl.ds(g * L, L)] = jnp.exp(jnp.maximum(x, x * 0.01))
            return c2
        lax.fori_loop(0, K // L, wgroup, 0)

        def edge(e, c2):
            wsp = plsc.load_gather(wbuf, [jnp.full((L,), 0, jnp.int32) + e])
            for j in range(d // L):
                rowbuf[e, pl.ds(j * L, L)] = rowbuf[e, pl.ds(j * L, L)] * wsp
            wtail[e, pl.ds(0, L)] = jnp.where(
                lane == 0, wsp,
                jnp.where(lane == 1, jnp.full((L,), 1.0, jnp.float32),
                          zero16))
            return c2
        lax.fori_loop(0, K, edge, 0)

        @pl.when(cid == 0)
        def _():
            pltpu.sync_copy(rowbuf, acc.at[hbuf], add=True)
            pltpu.sync_copy(wtail, acc2.at[hbuf], add=True)

        @pl.when(cid == 1)
        def _():
            pltpu.sync_copy(rowbuf, acc.at[rbuf], add=True)
            pltpu.sync_copy(wtail, acc2.at[rbuf], add=True)
        return c
    lax.fori_loop(0, nchunks, chunk, 0)

    plsc.subcore_barrier()

    @pl.when(cid == 0)
    def _():
        pltpu.sync_copy(acc.at[pl.ds(row0, rpt)], ha.at[pl.ds(row0, rpt)])
        pltpu.sync_copy(acc2.at[pl.ds(row0, rpt)], hw.at[pl.ds(row0, rpt)])

    @pl.when(cid == 1)
    def _():
        pltpu.sync_copy(acc.at[pl.ds(row0, rpt)], ra.at[pl.ds(row0, rpt)])
        pltpu.sync_copy(acc2.at[pl.ds(row0, rpt)], rw.at[pl.ds(row0, rpt)])


def kernel(triplets, ent_embed, rel_embed, a_w, a_b, a2_w, a2_b):
    n, d = ent_embed.shape
    r = rel_embed.shape[0]
    e = triplets.shape[0]
    f32 = jnp.float32

    np_ = ((max(n, r) + 1 + NS * 8 - 1) // (NS * 8)) * (NS * 8)
    et = ((e + NS * K - 1) // (NS * K)) * K
    ep = NS * et

    ent_p = jnp.pad(ent_embed.astype(f32), ((0, np_ - n), (0, 0)))
    rel_p = jnp.pad(rel_embed.astype(f32), ((0, np_ - r), (0, 0)))

    fill = jnp.broadcast_to(
        jnp.array([[n], [n], [r]], jnp.int32), (3, ep - e))
    tri_t = jnp.concatenate([triplets.T.astype(jnp.int32), fill], axis=1)

    (ph, pt, pr, s_h, s_t, s_r, hidx, tidx, ridx) = pl.pallas_call(
        _prep_body,
        out_shape=[
            jax.ShapeDtypeStruct((np_, d), f32),
            jax.ShapeDtypeStruct((np_, d), f32),
            jax.ShapeDtypeStruct((np_, d), f32),
            jax.ShapeDtypeStruct((np_,), f32),
            jax.ShapeDtypeStruct((np_,), f32),
            jax.ShapeDtypeStruct((np_,), f32),
            jax.ShapeDtypeStruct((ep,), jnp.int32),
            jax.ShapeDtypeStruct((ep,), jnp.int32),
            jax.ShapeDtypeStruct((ep,), jnp.int32),
        ],
    )(ent_p, rel_p, a_w, a_b.reshape(1, d), a2_w, a2_b.reshape(1, 1), tri_t)

    mesh = plsc.VectorSubcoreMesh(core_axis_name="c", subcore_axis_name="s",
                                  num_cores=NC, num_subcores=NS)
    sc_edge = pl.kernel(
        functools.partial(_sc_body, np_, et, d),
        out_type=[
            jax.ShapeDtypeStruct((np_, d), f32),
            jax.ShapeDtypeStruct((np_, L), f32),
            jax.ShapeDtypeStruct((np_, d), f32),
            jax.ShapeDtypeStruct((np_, L), f32),
        ],
        mesh=mesh,
        compiler_params=pltpu.CompilerParams(use_tc_tiling_on_sc=False,
                                             needs_layout_passes=False),
        scratch_types=[
            pltpu.VMEM((K,), jnp.int32),
            pltpu.VMEM((K,), jnp.int32),
            pltpu.VMEM((K,), jnp.int32),
            pltpu.VMEM((K, d), f32),
            pltpu.VMEM((K, L), f32),
            pltpu.VMEM((K,), f32),
            pltpu.VMEM((K,), f32),
            pltpu.VMEM((K,), f32),
            pltpu.VMEM((K,), f32),
            pltpu.VMEM_SHARED((np_, d), f32),
            pltpu.VMEM_SHARED((np_, L), f32),
            pltpu.SemaphoreType.DMA,
        ],
    )
    ha, hw, ra, rw = sc_edge(hidx, tidx, ridx, ph, pt, pr, s_h, s_t, s_r)

    he, hr = pl.pallas_call(
        functools.partial(_fin_body, n, r, d),
        out_shape=[
            jax.ShapeDtypeStruct((n, d), f32),
            jax.ShapeDtypeStruct((r, d), f32),
        ],
    )(ha, hw, ra, rw)
    return he, hr


# speedup vs baseline: 3.6176x; 1.4668x over previous
import functools

import jax
import jax.numpy as jnp
from jax import lax
from jax.experimental import pallas as pl
from jax.experimental.pallas import tpu as pltpu
from jax.experimental.pallas import tpu_sc as plsc

L = 16
NC = 2
NS = 16
K = 128
AW = 144


def _prep_body(ent_ref, rel_ref, aw_ref, ab_ref, a2w_ref, a2b_ref, tri_ref,
               ph_ref, pt_ref, pr_ref, sh_ref, st_ref, sr_ref,
               hi_ref, ti_ref, ri_ref):
    d = aw_ref.shape[0]
    np_ = ent_ref.shape[0]
    aw = aw_ref[...]
    ent = ent_ref[...]
    rel = rel_ref[...]
    ph = jnp.dot(ent, aw[:, 0:d].T, preferred_element_type=jnp.float32)
    pt = jnp.dot(ent, aw[:, d:2 * d].T, preferred_element_type=jnp.float32)
    pr = (jnp.dot(rel, aw[:, 2 * d:3 * d].T,
                  preferred_element_type=jnp.float32) + ab_ref[...])
    a2 = a2w_ref[...]
    sh = jnp.dot(a2, ph.T, preferred_element_type=jnp.float32)
    st = jnp.dot(a2, pt.T, preferred_element_type=jnp.float32)
    sr = (jnp.dot(a2, pr.T, preferred_element_type=jnp.float32)
          + a2b_ref[...])
    ph_ref[...] = ph
    pt_ref[...] = pt
    pr_ref[...] = pr
    sh_ref[...] = sh.reshape(np_)
    st_ref[...] = st.reshape(np_)
    sr_ref[...] = sr.reshape(np_)
    tri = tri_ref[...]
    hi_ref[...] = tri[0]
    ti_ref[...] = tri[1]
    ri_ref[...] = tri[2]


def _fin_body(n, r, d, ha_ref, hw_ref, ra_ref, rw_ref, ph_ref, pr_ref,
              he_ref, hr_ref):
    wsum_h = hw_ref[...][:n, 0:1]
    hsum = ha_ref[...][:n, :d] + wsum_h * ph_ref[...][:n, :]
    ebs = jnp.where(wsum_h == 0.0, 1e-12, wsum_h)
    he = hsum / ebs
    he_ref[...] = jnp.where(he > 0.0, he, jnp.exp(he) - 1.0)
    wsum_r = rw_ref[...][:r, 0:1]
    rsum = ra_ref[...][:r, :d] + wsum_r * pr_ref[...][:r, :]
    cnt = rw_ref[...][:r, 1:2]
    hr = rsum / jnp.maximum(cnt, 1.0)
    hr_ref[...] = jnp.where(hr > 0.0, hr, jnp.exp(hr) - 1.0)


def _sc_body(np_, et, d, hidx, tidx, ridx, ph, pt, pr, sh_h, st_h, sr_h,
             ha, hw, ra, rw,
             hbuf, tbuf, rbuf, rowbuf, wtail, wbuf, sgh, sgt, sgr,
             acc, acc2, sem, semS):
    cid = lax.axis_index("c")
    sid = lax.axis_index("s")
    zero16 = jnp.zeros((L,), jnp.float32)

    def zrow(i, c):
        for j in range(d // L):
            rowbuf[i, pl.ds(j * L, L)] = zero16
        wtail[i, pl.ds(0, L)] = zero16
        return c
    lax.fori_loop(0, K, zrow, 0)

    rpt = np_ // NS
    row0 = sid * rpt
    nfull = rpt // K
    for b in range(nfull):
        pltpu.sync_copy(rowbuf, acc.at[pl.ds(row0 + b * K, K)])
        pltpu.sync_copy(wtail, acc2.at[pl.ds(row0 + b * K, K)])
    rem = rpt - nfull * K
    if rem:
        pltpu.sync_copy(rowbuf.at[pl.ds(0, rem)],
                        acc.at[pl.ds(row0 + nfull * K, rem)])
        pltpu.sync_copy(wtail.at[pl.ds(0, rem)],
                        acc2.at[pl.ds(row0 + nfull * K, rem)])
    plsc.subcore_barrier()

    nchunks = et // K
    lane = lax.iota(jnp.int32, L)

    def chunk(ci, c):
        base = sid * et + ci * K
        pltpu.sync_copy(hidx.at[pl.ds(base, K)], hbuf)
        pltpu.sync_copy(tidx.at[pl.ds(base, K)], tbuf)
        pltpu.sync_copy(ridx.at[pl.ds(base, K)], rbuf)
        d1 = pltpu.async_copy(sh_h.at[hbuf], sgh, semS)
        d2 = pltpu.async_copy(st_h.at[tbuf], sgt, semS)
        d3 = pltpu.async_copy(sr_h.at[rbuf], sgr, semS)
        @pl.when(cid == 0)
        def _():
            pltpu.async_copy(pt.at[tbuf], rowbuf, sem).wait()
            pltpu.async_copy(pr.at[rbuf], rowbuf, sem, add=True).wait()

        @pl.when(cid == 1)
        def _():
            pltpu.async_copy(ph.at[hbuf], rowbuf, sem).wait()
            pltpu.async_copy(pt.at[tbuf], rowbuf, sem, add=True).wait()
        d1.wait()
        d2.wait()
        d3.wait()

        def wgroup(g, c2):
            x = (sgh[pl.ds(g * L, L)] + sgt[pl.ds(g * L, L)]
                 + sgr[pl.ds(g * L, L)])
            wbuf[pl.ds(g * L, L)] = jnp.exp(jnp.maximum(x, x * 0.01))
            return c2
        lax.fori_loop(0, K // L, wgroup, 0)

        def edge(e, c2):
            wsp = plsc.load_gather(wbuf, [jnp.full((L,), 0, jnp.int32) + e])
            for j in range(d // L):
                rowbuf[e, pl.ds(j * L, L)] = rowbuf[e, pl.ds(j * L, L)] * wsp
            wtail[e, pl.ds(0, L)] = jnp.where(
                lane == 0, wsp,
                jnp.where(lane == 1, jnp.full((L,), 1.0, jnp.float32),
                          zero16))
            return c2
        lax.fori_loop(0, K, edge, 0)

        @pl.when(cid == 0)
        def _():
            pltpu.sync_copy(rowbuf, acc.at[hbuf], add=True)
            pltpu.sync_copy(wtail, acc2.at[hbuf], add=True)

        @pl.when(cid == 1)
        def _():
            pltpu.sync_copy(rowbuf, acc.at[rbuf], add=True)
            pltpu.sync_copy(wtail, acc2.at[rbuf], add=True)
        return c
    lax.fori_loop(0, nchunks, chunk, 0)

    plsc.subcore_barrier()

    @pl.when(cid == 0)
    def _():
        pltpu.sync_copy(acc.at[pl.ds(row0, rpt)], ha.at[pl.ds(row0, rpt)])
        pltpu.sync_copy(acc2.at[pl.ds(row0, rpt)], hw.at[pl.ds(row0, rpt)])

    @pl.when(cid == 1)
    def _():
        pltpu.sync_copy(acc.at[pl.ds(row0, rpt)], ra.at[pl.ds(row0, rpt)])
        pltpu.sync_copy(acc2.at[pl.ds(row0, rpt)], rw.at[pl.ds(row0, rpt)])


def kernel(triplets, ent_embed, rel_embed, a_w, a_b, a2_w, a2_b):
    n, d = ent_embed.shape
    r = rel_embed.shape[0]
    e = triplets.shape[0]
    f32 = jnp.float32

    np_ = ((max(n, r) + 1 + NS * 8 - 1) // (NS * 8)) * (NS * 8)
    et = ((e + NS * K - 1) // (NS * K)) * K
    ep = NS * et

    ent_p = jnp.pad(ent_embed.astype(f32), ((0, np_ - n), (0, 0)))
    rel_p = jnp.pad(rel_embed.astype(f32), ((0, np_ - r), (0, 0)))

    fill = jnp.broadcast_to(
        jnp.array([[n], [n], [r]], jnp.int32), (3, ep - e))
    tri_t = jnp.concatenate([triplets.T.astype(jnp.int32), fill], axis=1)

    (ph, pt, pr, s_h, s_t, s_r, hidx, tidx, ridx) = pl.pallas_call(
        _prep_body,
        out_shape=[
            jax.ShapeDtypeStruct((np_, d), f32),
            jax.ShapeDtypeStruct((np_, d), f32),
            jax.ShapeDtypeStruct((np_, d), f32),
            jax.ShapeDtypeStruct((np_,), f32),
            jax.ShapeDtypeStruct((np_,), f32),
            jax.ShapeDtypeStruct((np_,), f32),
            jax.ShapeDtypeStruct((ep,), jnp.int32),
            jax.ShapeDtypeStruct((ep,), jnp.int32),
            jax.ShapeDtypeStruct((ep,), jnp.int32),
        ],
    )(ent_p, rel_p, a_w, a_b.reshape(1, d), a2_w, a2_b.reshape(1, 1), tri_t)

    mesh = plsc.VectorSubcoreMesh(core_axis_name="c", subcore_axis_name="s",
                                  num_cores=NC, num_subcores=NS)
    sc_edge = pl.kernel(
        functools.partial(_sc_body, np_, et, d),
        out_type=[
            jax.ShapeDtypeStruct((np_, d), f32),
            jax.ShapeDtypeStruct((np_, L), f32),
            jax.ShapeDtypeStruct((np_, d), f32),
            jax.ShapeDtypeStruct((np_, L), f32),
        ],
        mesh=mesh,
        compiler_params=pltpu.CompilerParams(use_tc_tiling_on_sc=False,
                                             needs_layout_passes=False),
        scratch_types=[
            pltpu.VMEM((K,), jnp.int32),
            pltpu.VMEM((K,), jnp.int32),
            pltpu.VMEM((K,), jnp.int32),
            pltpu.VMEM((K, d), f32),
            pltpu.VMEM((K, L), f32),
            pltpu.VMEM((K,), f32),
            pltpu.VMEM((K,), f32),
            pltpu.VMEM((K,), f32),
            pltpu.VMEM((K,), f32),
            pltpu.VMEM_SHARED((np_, d), f32),
            pltpu.VMEM_SHARED((np_, L), f32),
            pltpu.SemaphoreType.DMA,
            pltpu.SemaphoreType.DMA,
        ],
    )
    ha, hw, ra, rw = sc_edge(hidx, tidx, ridx, ph, pt, pr, s_h, s_t, s_r)

    he, hr = pl.pallas_call(
        functools.partial(_fin_body, n, r, d),
        out_shape=[
            jax.ShapeDtypeStruct((n, d), f32),
            jax.ShapeDtypeStruct((r, d), f32),
        ],
    )(ha, hw, ra, rw, ph, pr)
    return he, hr


# speedup vs baseline: 3.7844x; 1.0461x over previous
import functools

import jax
import jax.numpy as jnp
from jax import lax
from jax.experimental import pallas as pl
from jax.experimental.pallas import tpu as pltpu
from jax.experimental.pallas import tpu_sc as plsc

L = 16
NC = 2
NS = 16
K = 128
AW = 144


def _prep_body(ent_ref, rel_ref, aw_ref, ab_ref, a2w_ref, a2b_ref, tri_ref,
               ph_ref, pt_ref, pr_ref, sh_ref, st_ref, sr_ref,
               hi_ref, ti_ref, ri_ref):
    d = aw_ref.shape[0]
    np_ = ent_ref.shape[0]
    aw = aw_ref[...]
    ent = ent_ref[...]
    rel = rel_ref[...]
    ph = jnp.dot(ent, aw[:, 0:d].T, preferred_element_type=jnp.float32)
    pt = jnp.dot(ent, aw[:, d:2 * d].T, preferred_element_type=jnp.float32)
    pr = (jnp.dot(rel, aw[:, 2 * d:3 * d].T,
                  preferred_element_type=jnp.float32) + ab_ref[...])
    a2 = a2w_ref[...]
    sh = jnp.dot(a2, ph.T, preferred_element_type=jnp.float32)
    st = jnp.dot(a2, pt.T, preferred_element_type=jnp.float32)
    sr = (jnp.dot(a2, pr.T, preferred_element_type=jnp.float32)
          + a2b_ref[...])
    ph_ref[...] = ph
    pt_ref[...] = pt
    pr_ref[...] = pr
    sh_ref[...] = sh.reshape(np_)
    st_ref[...] = st.reshape(np_)
    sr_ref[...] = sr.reshape(np_)
    tri = tri_ref[...]
    hi_ref[...] = tri[0]
    ti_ref[...] = tri[1]
    ri_ref[...] = tri[2]


def _fin_body(n, r, d, ha_ref, hw_ref, ra_ref, rw_ref, ph_ref, pr_ref,
              he_ref, hr_ref):
    wsum_h = hw_ref[...][:n, 0:1]
    hsum = ha_ref[...][:n, :d] + wsum_h * ph_ref[...][:n, :]
    ebs = jnp.where(wsum_h == 0.0, 1e-12, wsum_h)
    he = hsum / ebs
    he_ref[...] = jnp.where(he > 0.0, he, jnp.exp(he) - 1.0)
    wsum_r = rw_ref[...][:r, 0:1]
    rsum = ra_ref[...][:r, :d] + wsum_r * pr_ref[...][:r, :]
    cnt = rw_ref[...][:r, 1:2]
    hr = rsum / jnp.maximum(cnt, 1.0)
    hr_ref[...] = jnp.where(hr > 0.0, hr, jnp.exp(hr) - 1.0)


def _sc_body(np_, et, d, hidx, tidx, ridx, ph, pt, pr, sh_h, st_h, sr_h,
             ha, hw, ra, rw,
             hbuf0, tbuf0, rbuf0, hbuf1, tbuf1, rbuf1,
             rowbuf0, rowbuf1, wtail, wbuf,
             sgh0, sgt0, sgr0, sgh1, sgt1, sgr1,
             acc, acc2,
             semI0, semI1, semS0, semS1, semG0, semG1):
    cid = lax.axis_index("c")
    sid = lax.axis_index("s")
    zero16 = jnp.zeros((L,), jnp.float32)
    IDX = [(hbuf0, tbuf0, rbuf0, semI0), (hbuf1, tbuf1, rbuf1, semI1)]
    SG = [(sgh0, sgt0, sgr0, semS0), (sgh1, sgt1, sgr1, semS1)]
    ROW = [(rowbuf0, semG0), (rowbuf1, semG1)]

    def zrow(i, c):
        for j in range(d // L):
            rowbuf0[i, pl.ds(j * L, L)] = zero16
        wtail[i, pl.ds(0, L)] = zero16
        return c
    lax.fori_loop(0, K, zrow, 0)

    rpt = np_ // NS
    row0 = sid * rpt
    nfull = rpt // K
    for b in range(nfull):
        pltpu.sync_copy(rowbuf0, acc.at[pl.ds(row0 + b * K, K)])
        pltpu.sync_copy(wtail, acc2.at[pl.ds(row0 + b * K, K)])
    rem = rpt - nfull * K
    if rem:
        pltpu.sync_copy(rowbuf0.at[pl.ds(0, rem)],
                        acc.at[pl.ds(row0 + nfull * K, rem)])
        pltpu.sync_copy(wtail.at[pl.ds(0, rem)],
                        acc2.at[pl.ds(row0 + nfull * K, rem)])
    plsc.subcore_barrier()

    nchunks = et // K
    lane = lax.iota(jnp.int32, L)

    def ebase(ci):
        return sid * et + ci * K

    def issue_idx(ci, p):
        hb, tb, rb, sem = IDX[p]
        pltpu.async_copy(hidx.at[pl.ds(ebase(ci), K)], hb, sem)
        pltpu.async_copy(tidx.at[pl.ds(ebase(ci), K)], tb, sem)
        pltpu.async_copy(ridx.at[pl.ds(ebase(ci), K)], rb, sem)

    def wait_idx(ci, p):
        hb, tb, rb, sem = IDX[p]
        pltpu.make_async_copy(hidx.at[pl.ds(ebase(ci), K)], hb, sem).wait()
        pltpu.make_async_copy(tidx.at[pl.ds(ebase(ci), K)], tb, sem).wait()
        pltpu.make_async_copy(ridx.at[pl.ds(ebase(ci), K)], rb, sem).wait()

    def issue_sg(p):
        hb, tb, rb, _ = IDX[p]
        gh, gt, gr, sem = SG[p]
        pltpu.async_copy(sh_h.at[hb], gh, sem)
        pltpu.async_copy(st_h.at[tb], gt, sem)
        pltpu.async_copy(sr_h.at[rb], gr, sem)

    def wait_sg(p):
        hb, tb, rb, _ = IDX[p]
        gh, gt, gr, sem = SG[p]
        pltpu.make_async_copy(sh_h.at[hb], gh, sem).wait()
        pltpu.make_async_copy(st_h.at[tb], gt, sem).wait()
        pltpu.make_async_copy(sr_h.at[rb], gr, sem).wait()

    def issue_g1(p):
        hb, tb, rb, _ = IDX[p]
        row, sem = ROW[p]

        @pl.when(cid == 0)
        def _():
            pltpu.async_copy(pt.at[tb], row, sem)

        @pl.when(cid == 1)
        def _():
            pltpu.async_copy(ph.at[hb], row, sem)

    def wait_g(p):
        tb = IDX[p][1]
        row, sem = ROW[p]
        pltpu.make_async_copy(pt.at[tb], row, sem).wait()

    def issue_g2(p):
        hb, tb, rb, _ = IDX[p]
        row, sem = ROW[p]

        @pl.when(cid == 0)
        def _():
            pltpu.async_copy(pr.at[rb], row, sem, add=True)

        @pl.when(cid == 1)
        def _():
            pltpu.async_copy(pt.at[tb], row, sem, add=True)

    def body(ci, p):
        q = 1 - p
        row, _ = ROW[p]
        hb, tb, rb, _ = IDX[p]
        gh, gt, gr, _ = SG[p]
        wait_g(p)
        wait_sg(p)

        def wgroup(g, c2):
            x = (gh[pl.ds(g * L, L)] + gt[pl.ds(g * L, L)]
                 + gr[pl.ds(g * L, L)])
            wbuf[pl.ds(g * L, L)] = jnp.exp(jnp.maximum(x, x * 0.01))
            return c2
        lax.fori_loop(0, K // L, wgroup, 0)

        @pl.when(ci + 1 < nchunks)
        def _():
            wait_idx(ci + 1, q)
            issue_sg(q)
            issue_g1(q)

        def edge(e, c2):
            wsp = plsc.load_gather(wbuf, [jnp.full((L,), 0, jnp.int32) + e])
            for j in range(d // L):
                row[e, pl.ds(j * L, L)] = row[e, pl.ds(j * L, L)] * wsp
            wtail[e, pl.ds(0, L)] = jnp.where(
                lane == 0, wsp,
                jnp.where(lane == 1, jnp.full((L,), 1.0, jnp.float32),
                          zero16))
            return c2
        lax.fori_loop(0, K, edge, 0)

        @pl.when(ci + 1 < nchunks)
        def _():
            wait_g(q)
            issue_g2(q)

        @pl.when(cid == 0)
        def _():
            pltpu.sync_copy(row, acc.at[hb], add=True)
            pltpu.sync_copy(wtail, acc2.at[hb], add=True)

        @pl.when(cid == 1)
        def _():
            pltpu.sync_copy(row, acc.at[rb], add=True)
            pltpu.sync_copy(wtail, acc2.at[rb], add=True)

        @pl.when(ci + 2 < nchunks)
        def _():
            issue_idx(ci + 2, p)

    issue_idx(0, 0)
    wait_idx(0, 0)
    issue_sg(0)
    issue_g1(0)
    wait_g(0)
    issue_g2(0)
    issue_idx(1, 1)

    def pair(ci2, c):
        body(2 * ci2, 0)
        body(2 * ci2 + 1, 1)
        return c
    lax.fori_loop(0, nchunks // 2, pair, 0)

    plsc.subcore_barrier()

    @pl.when(cid == 0)
    def _():
        pltpu.sync_copy(acc.at[pl.ds(row0, rpt)], ha.at[pl.ds(row0, rpt)])
        pltpu.sync_copy(acc2.at[pl.ds(row0, rpt)], hw.at[pl.ds(row0, rpt)])

    @pl.when(cid == 1)
    def _():
        pltpu.sync_copy(acc.at[pl.ds(row0, rpt)], ra.at[pl.ds(row0, rpt)])
        pltpu.sync_copy(acc2.at[pl.ds(row0, rpt)], rw.at[pl.ds(row0, rpt)])


def kernel(triplets, ent_embed, rel_embed, a_w, a_b, a2_w, a2_b):
    n, d = ent_embed.shape
    r = rel_embed.shape[0]
    e = triplets.shape[0]
    f32 = jnp.float32

    np_ = ((max(n, r) + 1 + NS * 8 - 1) // (NS * 8)) * (NS * 8)
    et = ((e + NS * 2 * K - 1) // (NS * 2 * K)) * 2 * K
    ep = NS * et

    ent_p = jnp.pad(ent_embed.astype(f32), ((0, np_ - n), (0, 0)))
    rel_p = jnp.pad(rel_embed.astype(f32), ((0, np_ - r), (0, 0)))

    fill = jnp.broadcast_to(
        jnp.array([[n], [n], [r]], jnp.int32), (3, ep - e))
    tri_t = jnp.concatenate([triplets.T.astype(jnp.int32), fill], axis=1)

    (ph, pt, pr, s_h, s_t, s_r, hidx, tidx, ridx) = pl.pallas_call(
        _prep_body,
        out_shape=[
            jax.ShapeDtypeStruct((np_, d), f32),
            jax.ShapeDtypeStruct((np_, d), f32),
            jax.ShapeDtypeStruct((np_, d), f32),
            jax.ShapeDtypeStruct((np_,), f32),
            jax.ShapeDtypeStruct((np_,), f32),
            jax.ShapeDtypeStruct((np_,), f32),
            jax.ShapeDtypeStruct((ep,), jnp.int32),
            jax.ShapeDtypeStruct((ep,), jnp.int32),
            jax.ShapeDtypeStruct((ep,), jnp.int32),
        ],
    )(ent_p, rel_p, a_w, a_b.reshape(1, d), a2_w, a2_b.reshape(1, 1), tri_t)

    mesh = plsc.VectorSubcoreMesh(core_axis_name="c", subcore_axis_name="s",
                                  num_cores=NC, num_subcores=NS)
    sc_edge = pl.kernel(
        functools.partial(_sc_body, np_, et, d),
        out_type=[
            jax.ShapeDtypeStruct((np_, d), f32),
            jax.ShapeDtypeStruct((np_, L), f32),
            jax.ShapeDtypeStruct((np_, d), f32),
            jax.ShapeDtypeStruct((np_, L), f32),
        ],
        mesh=mesh,
        compiler_params=pltpu.CompilerParams(use_tc_tiling_on_sc=False,
                                             needs_layout_passes=False),
        scratch_types=(
            [pltpu.VMEM((K,), jnp.int32)] * 6
            + [pltpu.VMEM((K, d), f32)] * 2
            + [pltpu.VMEM((K, L), f32)]
            + [pltpu.VMEM((K,), f32)] * 7
            + [pltpu.VMEM_SHARED((np_, d), f32),
               pltpu.VMEM_SHARED((np_, L), f32)]
            + [pltpu.SemaphoreType.DMA] * 6
        ),
    )
    ha, hw, ra, rw = sc_edge(hidx, tidx, ridx, ph, pt, pr, s_h, s_t, s_r)

    he, hr = pl.pallas_call(
        functools.partial(_fin_body, n, r, d),
        out_shape=[
            jax.ShapeDtypeStruct((n, d), f32),
            jax.ShapeDtypeStruct((r, d), f32),
        ],
    )(ha, hw, ra, rw, ph, pr)
    return he, hr
